# baseline (speedup 1.0000x reference)
import jax
import jax.numpy as jnp
from jax.experimental import pallas as pl
from jax.experimental.pallas import tpu as pltpu

K = 100000
D = 1024
TOP_K = 32
TAU = 1.0

NUM_CHUNKS = 50
CHUNK = K // NUM_CHUNKS

INTERPRET = False


def _scores_topk_kernel(q_ref, k_ref, logits_ref, idx_ref, s_ref):
    i = pl.program_id(0)
    row = jax.lax.dot_general(
        q_ref[...], k_ref[...],
        dimension_numbers=(((1,), (1,)), ((), ())),
        preferred_element_type=jnp.float32,
    )
    s_ref[pl.ds(i, 1), :] = row

    @pl.when(i == NUM_CHUNKS - 1)
    def _():
        row_iota = jax.lax.broadcasted_iota(jnp.int32, (NUM_CHUNKS, CHUNK), 0)
        col_iota = jax.lax.broadcasted_iota(jnp.int32, (NUM_CHUNKS, CHUNK), 1)
        flat = row_iota * CHUNK + col_iota
        s = s_ref[...]
        neg_inf = jnp.float32(-jnp.inf)
        for j in range(TOP_K):
            m = jnp.max(s)
            pos = jnp.min(jnp.where(s == m, flat, jnp.int32(K)))
            logits_ref[j] = m
            idx_ref[j] = pos
            s = jnp.where(flat == pos, neg_inf, s)


def _gather_combine_kernel(idx_ref, k_ref, logits_ref, g_ref, o_ref, rows_ref):
    j = pl.program_id(0)
    rows_ref[pl.ds(j, 1), :] = k_ref[0]

    @pl.when(j == TOP_K - 1)
    def _():
        z = (logits_ref[...] + g_ref[...]) / jnp.float32(TAU)
        z = z - jnp.max(z)
        e = jnp.exp(z)
        w = e / jnp.sum(e)
        o_ref[...] = jax.lax.dot_general(
            w, rows_ref[...],
            dimension_numbers=(((1,), (0,)), ((), ())),
            preferred_element_type=jnp.float32,
        )


def kernel(query_vec, keys):
    q = query_vec.reshape(1, D)

    logits, idx = pl.pallas_call(
        _scores_topk_kernel,
        grid=(NUM_CHUNKS,),
        in_specs=[
            pl.BlockSpec((1, D), lambda i: (0, 0)),
            pl.BlockSpec((CHUNK, D), lambda i: (i, 0)),
        ],
        out_specs=[
            pl.BlockSpec(memory_space=pltpu.SMEM),
            pl.BlockSpec(memory_space=pltpu.SMEM),
        ],
        out_shape=[
            jax.ShapeDtypeStruct((TOP_K,), jnp.float32),
            jax.ShapeDtypeStruct((TOP_K,), jnp.int32),
        ],
        scratch_shapes=[pltpu.VMEM((NUM_CHUNKS, CHUNK), jnp.float32)],
        interpret=INTERPRET,
    )(q, keys)

    u = jax.random.uniform(jax.random.key(42), (TOP_K,),
                           minval=1e-6, maxval=1.0 - 1e-6)
    g = (-jnp.log(-jnp.log(u))).reshape(1, TOP_K)

    out = pl.pallas_call(
        _gather_combine_kernel,
        grid_spec=pltpu.PrefetchScalarGridSpec(
            num_scalar_prefetch=1,
            grid=(TOP_K,),
            in_specs=[
                pl.BlockSpec((1, 1, D), lambda j, idx_ref: (idx_ref[j], 0, 0)),
                pl.BlockSpec((1, TOP_K), lambda j, idx_ref: (0, 0)),
                pl.BlockSpec((1, TOP_K), lambda j, idx_ref: (0, 0)),
            ],
            out_specs=pl.BlockSpec((1, D), lambda j, idx_ref: (0, 0)),
            scratch_shapes=[pltpu.VMEM((TOP_K, D), jnp.float32)],
        ),
        out_shape=jax.ShapeDtypeStruct((1, D), jnp.float32),
        interpret=INTERPRET,
    )(idx, keys.reshape(K, 1, D), logits.reshape(1, TOP_K), g)

    return out.reshape(D), jnp.arange(TOP_K)
